# baseline (speedup 1.0000x reference)
import functools

import jax
import jax.numpy as jnp
from jax import lax
from jax.experimental import pallas as pl
from jax.experimental.pallas import tpu as pltpu
from jax.experimental.pallas import tpu_sc as plsc

_PRED_T = 0.3
_TRUE_T = 0.5
_PW = 0.05
_HB = 56
_RPW = 28
_NW = 32


def _pred_body(p_ref, out_ref, yb):
    h = pl.program_id(1)
    nh = pl.num_programs(1)
    HB, W = p_ref.shape[1], p_ref.shape[3]
    H = HB * nh
    f32 = jnp.float32

    p = p_ref[0]
    pm = jnp.max(p, axis=1)
    prow = jnp.max(pm, axis=1)
    pcol = jnp.max(pm, axis=0)

    hidx = jax.lax.broadcasted_iota(jnp.int32, (HB,), 0).astype(f32) + jnp.float32(
        h * HB
    )
    widx = jax.lax.broadcasted_iota(jnp.int32, (W,), 0).astype(f32)

    first = h == 0
    pymin = jnp.min(jnp.where(prow > _PRED_T, hidx, jnp.float32(H)))
    pymax = jnp.max(jnp.where(prow > _PRED_T, hidx, -1.0))
    pxmin = jnp.min(jnp.where(pcol > _PRED_T, widx, jnp.float32(W)))
    pxmax = jnp.max(jnp.where(pcol > _PRED_T, widx, -1.0))

    yb[0] = jnp.minimum(jnp.where(first, jnp.float32(H), yb[0]), pymin)
    yb[1] = jnp.maximum(jnp.where(first, -1.0, yb[1]), pymax)
    yb[2] = jnp.minimum(jnp.where(first, jnp.float32(W), yb[2]), pxmin)
    yb[3] = jnp.maximum(jnp.where(first, -1.0, yb[3]), pxmax)

    @pl.when(h == nh - 1)
    def _tail():
        lane = jax.lax.broadcasted_iota(jnp.int32, (1, 128), 1)
        v = jnp.where(
            lane == 0,
            jnp.full((1, 128), yb[0], f32),
            jnp.where(
                lane == 1,
                jnp.full((1, 128), yb[1], f32),
                jnp.where(
                    lane == 2,
                    jnp.full((1, 128), yb[2], f32),
                    jnp.full((1, 128), yb[3], f32),
                ),
            ),
        )
        out_ref[0] = v


def _sc_body(x_hbm, outr_hbm, outc_hbm, buf0, buf1, acc_col, acc_row, sem0, sem1):
    f32 = jnp.float32
    wid = lax.axis_index("s") * 2 + lax.axis_index("c")
    b = wid // 8
    h0 = (wid % 8) * _RPW

    neg = jnp.full((16,), -3.0e38, f32)
    for j in range(14):
        acc_col[pl.ds(16 * j, 16)] = neg

    def start(i, buf, sem):
        @pl.when(i < _RPW)
        def _():
            pltpu.make_async_copy(x_hbm.at[b, h0 + i], buf, sem).start()

    def wait(buf, sem):
        pltpu.make_async_copy(x_hbm.at[b, h0], buf, sem).wait()

    start(0, buf0, sem0)
    start(1, buf1, sem1)

    lanes16 = jax.lax.broadcasted_iota(jnp.int32, (16,), 0)

    def process(i, buf, rv0, rv1):
        sls = [pl.ds(16 * j, 16) for j in range(14)]
        parts = [buf[0, sl] for sl in sls]
        for c in range(1, 96):
            for j in range(14):
                parts[j] = jnp.maximum(parts[j], buf[c, sls[j]])
        for j in range(14):
            acc_col[sls[j]] = jnp.maximum(acc_col[sls[j]], parts[j])
        rm = parts[0]
        for j in range(1, 14):
            rm = jnp.maximum(rm, parts[j])
        mv = jnp.full((16,), jnp.max(rm), f32)
        iv = jnp.full((16,), i, jnp.int32)
        rv0 = jnp.where(lanes16 == iv, mv, rv0)
        rv1 = jnp.where(lanes16 == iv - 16, mv, rv1)
        return rv0, rv1

    def body(t, carry):
        rv0, rv1 = carry
        i = 2 * t
        wait(buf0, sem0)
        rv0, rv1 = process(i, buf0, rv0, rv1)
        start(i + 2, buf0, sem0)
        wait(buf1, sem1)
        rv0, rv1 = process(i + 1, buf1, rv0, rv1)
        start(i + 3, buf1, sem1)
        return rv0, rv1

    rv0, rv1 = lax.fori_loop(0, _RPW // 2, body, (neg, neg))
    acc_row[pl.ds(0, 16)] = rv0
    acc_row[pl.ds(16, 16)] = rv1

    pltpu.sync_copy(acc_row, outr_hbm.at[wid])
    pltpu.sync_copy(acc_col, outc_hbm.at[wid])


def _combine_body(pb_ref, rows_ref, cols_ref, out_ref):
    f32 = jnp.float32
    rm = rows_ref[...]
    cm = cols_ref[...]
    pbv = pb_ref[...]

    wi_r = jax.lax.broadcasted_iota(jnp.int32, (_NW, 32), 0)
    ii_r = jax.lax.broadcasted_iota(jnp.int32, (_NW, 32), 1)
    hw = ((wi_r % 8) * _RPW + ii_r).astype(f32)
    bidx_r = wi_r // 8
    valid_r = ii_r < _RPW
    wi_c = jax.lax.broadcasted_iota(jnp.int32, (_NW, 224), 0)
    wv_c = jax.lax.broadcasted_iota(jnp.int32, (_NW, 224), 1).astype(f32)
    bidx_c = wi_c // 8
    lane = jax.lax.broadcasted_iota(jnp.int32, (1, 128), 1)

    def vec(s):
        return jnp.full((1, 128), s, f32)

    def pick(row, k):
        return jnp.max(jnp.where(lane[0] == k, row, -3.0e38))

    tot = jnp.zeros((1, 128), f32)
    for b in range(4):
        selr = (bidx_r == b) & valid_r & (rm > _TRUE_T)
        ty1 = jnp.min(jnp.where(selr, hw, 224.0))
        ty2 = jnp.max(jnp.where(selr, hw, -1.0))
        selc = (bidx_c == b) & (cm > _TRUE_T)
        tx1 = jnp.min(jnp.where(selc, wv_c, 224.0))
        tx2 = jnp.max(jnp.where(selc, wv_c, -1.0))
        row = pbv[b, 0]
        py1 = pick(row, 0)
        py2 = pick(row, 1)
        px1 = pick(row, 2)
        px2 = pick(row, 3)
        pa = vec((py2 - py1 + 1.0) * (px2 - px1 + 1.0))
        ta = vec((ty2 - ty1 + 1.0) * (tx2 - tx1 + 1.0))
        area_pen = jnp.maximum(pa - ta, 0.0) / (ta + 1.0)
        cy = vec(py1 + py2) * 0.5 - vec(ty1 + ty2) * 0.5
        cx = vec(px1 + px2) * 0.5 - vec(tx1 + tx2) * 0.5
        center = jnp.sqrt(cy * cy + cx * cx) * (1.0 / 20.0)
        valid = jnp.full((1, 128), (py2 >= 0.0) & (ty2 >= 0.0), jnp.bool_)
        pen = jnp.where(valid, area_pen + center, 1.0)
        tot = tot + pen
    out_ref[...] = tot * (_PW / 4.0)


def kernel(prediction_probs, expected_onehot):
    B, H, W, C = prediction_probs.shape
    pt = prediction_probs.transpose(0, 1, 3, 2)
    et = expected_onehot.transpose(0, 1, 3, 2)
    nh = H // _HB

    predbb = pl.pallas_call(
        _pred_body,
        grid=(B, nh),
        in_specs=[pl.BlockSpec((1, _HB, C, W), lambda b, h: (b, h, 0, 0))],
        out_specs=pl.BlockSpec((1, 1, 128), lambda b, h: (b, 0, 0)),
        out_shape=jax.ShapeDtypeStruct((B, 1, 128), jnp.float32),
        scratch_shapes=[pltpu.SMEM((4,), jnp.float32)],
    )(pt)

    sc = pl.kernel(
        _sc_body,
        out_type=(
            jax.ShapeDtypeStruct((_NW, 32), jnp.float32),
            jax.ShapeDtypeStruct((_NW, 224), jnp.float32),
        ),
        mesh=plsc.VectorSubcoreMesh(core_axis_name="c", subcore_axis_name="s"),
        compiler_params=pltpu.CompilerParams(needs_layout_passes=False),
        scratch_types=[
            pltpu.VMEM((96, 224), jnp.float32),
            pltpu.VMEM((96, 224), jnp.float32),
            pltpu.VMEM((224,), jnp.float32),
            pltpu.VMEM((32,), jnp.float32),
            pltpu.SemaphoreType.DMA,
            pltpu.SemaphoreType.DMA,
        ],
    )
    erows, ecols = sc(et)

    out = pl.pallas_call(
        _combine_body,
        out_shape=jax.ShapeDtypeStruct((1, 128), jnp.float32),
    )(predbb, erows, ecols)
    return out[0, 0]


# speedup vs baseline: 2.7018x; 2.7018x over previous
import functools

import jax
import jax.numpy as jnp
from jax import lax
from jax.experimental import pallas as pl
from jax.experimental.pallas import tpu as pltpu
from jax.experimental.pallas import tpu_sc as plsc

_PRED_T = 0.3
_TRUE_T = 0.5
_PW = 0.05
_HB = 56
_RPW = 28
_NW = 32


def _pred_body(p_ref, out_ref, yb):
    h = pl.program_id(1)
    nh = pl.num_programs(1)
    HB, W = p_ref.shape[1], p_ref.shape[3]
    H = HB * nh
    f32 = jnp.float32

    p = p_ref[0]
    pm = jnp.max(p, axis=1)
    prow = jnp.max(pm, axis=1)
    pcol = jnp.max(pm, axis=0)

    hidx = jax.lax.broadcasted_iota(jnp.int32, (HB,), 0).astype(f32) + jnp.float32(
        h * HB
    )
    widx = jax.lax.broadcasted_iota(jnp.int32, (W,), 0).astype(f32)

    first = h == 0
    pymin = jnp.min(jnp.where(prow > _PRED_T, hidx, jnp.float32(H)))
    pymax = jnp.max(jnp.where(prow > _PRED_T, hidx, -1.0))
    pxmin = jnp.min(jnp.where(pcol > _PRED_T, widx, jnp.float32(W)))
    pxmax = jnp.max(jnp.where(pcol > _PRED_T, widx, -1.0))

    yb[0] = jnp.minimum(jnp.where(first, jnp.float32(H), yb[0]), pymin)
    yb[1] = jnp.maximum(jnp.where(first, -1.0, yb[1]), pymax)
    yb[2] = jnp.minimum(jnp.where(first, jnp.float32(W), yb[2]), pxmin)
    yb[3] = jnp.maximum(jnp.where(first, -1.0, yb[3]), pxmax)

    @pl.when(h == nh - 1)
    def _tail():
        lane = jax.lax.broadcasted_iota(jnp.int32, (1, 128), 1)
        v = jnp.where(
            lane == 0,
            jnp.full((1, 128), yb[0], f32),
            jnp.where(
                lane == 1,
                jnp.full((1, 128), yb[1], f32),
                jnp.where(
                    lane == 2,
                    jnp.full((1, 128), yb[2], f32),
                    jnp.full((1, 128), yb[3], f32),
                ),
            ),
        )
        out_ref[0] = v


def _sc_body(x_hbm, outr_hbm, outc_hbm, buf0, buf1, acc_col, acc_row, sem0, sem1):
    f32 = jnp.float32
    wid = lax.axis_index("s") * 2 + lax.axis_index("c")
    b = wid // 8
    h0 = (wid % 8) * _RPW

    neg = jnp.full((16,), -3.0e38, f32)
    for j in range(14):
        acc_col[pl.ds(16 * j, 16)] = neg

    def start(i, buf, sem):
        @pl.when(i < _RPW)
        def _():
            pltpu.make_async_copy(x_hbm.at[b, h0 + i], buf, sem).start()

    def wait(buf, sem):
        pltpu.make_async_copy(x_hbm.at[b, h0], buf, sem).wait()

    start(0, buf0, sem0)
    start(1, buf1, sem1)

    lanes16 = jax.lax.broadcasted_iota(jnp.int32, (16,), 0)

    def process(i, buf, rv0, rv1):
        sls = [pl.ds(16 * j, 16) for j in range(14)]
        parts = [buf[0, sl] for sl in sls]
        for c in range(1, 2):
            for j in range(14):
                parts[j] = jnp.maximum(parts[j], buf[c, sls[j]])
        for j in range(14):
            acc_col[sls[j]] = jnp.maximum(acc_col[sls[j]], parts[j])
        rm = parts[0]
        for j in range(1, 14):
            rm = jnp.maximum(rm, parts[j])
        mv = jnp.full((16,), jnp.max(rm), f32)
        iv = jnp.full((16,), i, jnp.int32)
        rv0 = jnp.where(lanes16 == iv, mv, rv0)
        rv1 = jnp.where(lanes16 == iv - 16, mv, rv1)
        return rv0, rv1

    def body(t, carry):
        rv0, rv1 = carry
        i = 2 * t
        wait(buf0, sem0)
        rv0, rv1 = process(i, buf0, rv0, rv1)
        start(i + 2, buf0, sem0)
        wait(buf1, sem1)
        rv0, rv1 = process(i + 1, buf1, rv0, rv1)
        start(i + 3, buf1, sem1)
        return rv0, rv1

    rv0, rv1 = lax.fori_loop(0, _RPW // 2, body, (neg, neg))
    acc_row[pl.ds(0, 16)] = rv0
    acc_row[pl.ds(16, 16)] = rv1

    pltpu.sync_copy(acc_row, outr_hbm.at[wid])
    pltpu.sync_copy(acc_col, outc_hbm.at[wid])


def _combine_body(pb_ref, rows_ref, cols_ref, out_ref):
    f32 = jnp.float32
    rm = rows_ref[...]
    cm = cols_ref[...]
    pbv = pb_ref[...]

    wi_r = jax.lax.broadcasted_iota(jnp.int32, (_NW, 32), 0)
    ii_r = jax.lax.broadcasted_iota(jnp.int32, (_NW, 32), 1)
    hw = ((wi_r % 8) * _RPW + ii_r).astype(f32)
    bidx_r = wi_r // 8
    valid_r = ii_r < _RPW
    wi_c = jax.lax.broadcasted_iota(jnp.int32, (_NW, 224), 0)
    wv_c = jax.lax.broadcasted_iota(jnp.int32, (_NW, 224), 1).astype(f32)
    bidx_c = wi_c // 8
    lane = jax.lax.broadcasted_iota(jnp.int32, (1, 128), 1)

    def vec(s):
        return jnp.full((1, 128), s, f32)

    def pick(row, k):
        return jnp.max(jnp.where(lane[0] == k, row, -3.0e38))

    tot = jnp.zeros((1, 128), f32)
    for b in range(4):
        selr = (bidx_r == b) & valid_r & (rm > _TRUE_T)
        ty1 = jnp.min(jnp.where(selr, hw, 224.0))
        ty2 = jnp.max(jnp.where(selr, hw, -1.0))
        selc = (bidx_c == b) & (cm > _TRUE_T)
        tx1 = jnp.min(jnp.where(selc, wv_c, 224.0))
        tx2 = jnp.max(jnp.where(selc, wv_c, -1.0))
        row = pbv[b, 0]
        py1 = pick(row, 0)
        py2 = pick(row, 1)
        px1 = pick(row, 2)
        px2 = pick(row, 3)
        pa = vec((py2 - py1 + 1.0) * (px2 - px1 + 1.0))
        ta = vec((ty2 - ty1 + 1.0) * (tx2 - tx1 + 1.0))
        area_pen = jnp.maximum(pa - ta, 0.0) / (ta + 1.0)
        cy = vec(py1 + py2) * 0.5 - vec(ty1 + ty2) * 0.5
        cx = vec(px1 + px2) * 0.5 - vec(tx1 + tx2) * 0.5
        center = jnp.sqrt(cy * cy + cx * cx) * (1.0 / 20.0)
        valid = jnp.full((1, 128), (py2 >= 0.0) & (ty2 >= 0.0), jnp.bool_)
        pen = jnp.where(valid, area_pen + center, 1.0)
        tot = tot + pen
    out_ref[...] = tot * (_PW / 4.0)


def kernel(prediction_probs, expected_onehot):
    B, H, W, C = prediction_probs.shape
    pt = prediction_probs.transpose(0, 1, 3, 2)
    et = expected_onehot.transpose(0, 1, 3, 2)
    nh = H // _HB

    predbb = pl.pallas_call(
        _pred_body,
        grid=(B, nh),
        in_specs=[pl.BlockSpec((1, _HB, C, W), lambda b, h: (b, h, 0, 0))],
        out_specs=pl.BlockSpec((1, 1, 128), lambda b, h: (b, 0, 0)),
        out_shape=jax.ShapeDtypeStruct((B, 1, 128), jnp.float32),
        scratch_shapes=[pltpu.SMEM((4,), jnp.float32)],
    )(pt)

    sc = pl.kernel(
        _sc_body,
        out_type=(
            jax.ShapeDtypeStruct((_NW, 32), jnp.float32),
            jax.ShapeDtypeStruct((_NW, 224), jnp.float32),
        ),
        mesh=plsc.VectorSubcoreMesh(core_axis_name="c", subcore_axis_name="s"),
        compiler_params=pltpu.CompilerParams(needs_layout_passes=False),
        scratch_types=[
            pltpu.VMEM((96, 224), jnp.float32),
            pltpu.VMEM((96, 224), jnp.float32),
            pltpu.VMEM((224,), jnp.float32),
            pltpu.VMEM((32,), jnp.float32),
            pltpu.SemaphoreType.DMA,
            pltpu.SemaphoreType.DMA,
        ],
    )
    erows, ecols = sc(et)

    out = pl.pallas_call(
        _combine_body,
        out_shape=jax.ShapeDtypeStruct((1, 128), jnp.float32),
    )(predbb, erows, ecols)
    return out[0, 0]


# speedup vs baseline: 3.8046x; 1.4082x over previous
import jax
import jax.numpy as jnp
from jax.experimental import pallas as pl
from jax.experimental.pallas import tpu as pltpu

_PRED_T = 0.3
_TRUE_T = 0.5
_PW = 0.05
_HB = 56


def _bbox_body(p_ref, e_ref, out_ref, yb, psum_ref):
    b = pl.program_id(0)
    h = pl.program_id(1)
    nb = pl.num_programs(0)
    nh = pl.num_programs(1)
    HB, W = p_ref.shape[1], p_ref.shape[3]
    H = HB * nh
    f32 = jnp.float32

    p = p_ref[0]
    e = e_ref[0]
    pm = jnp.max(p, axis=1)
    em = jnp.max(e, axis=1)
    prow = jnp.max(pm, axis=1)
    erow = jnp.max(em, axis=1)
    pcol = jnp.max(pm, axis=0)
    ecol = jnp.max(em, axis=0)

    hidx = jax.lax.broadcasted_iota(jnp.int32, (HB,), 0).astype(f32) + jnp.float32(
        h * HB
    )
    widx = jax.lax.broadcasted_iota(jnp.int32, (W,), 0).astype(f32)

    fH = jnp.float32(H)
    fW = jnp.float32(W)
    first = h == 0
    pymin = jnp.min(jnp.where(prow > _PRED_T, hidx, fH))
    pymax = jnp.max(jnp.where(prow > _PRED_T, hidx, -1.0))
    pxmin = jnp.min(jnp.where(pcol > _PRED_T, widx, fW))
    pxmax = jnp.max(jnp.where(pcol > _PRED_T, widx, -1.0))
    tymin = jnp.min(jnp.where(erow > _TRUE_T, hidx, fH))
    tymax = jnp.max(jnp.where(erow > _TRUE_T, hidx, -1.0))
    txmin = jnp.min(jnp.where(ecol > _TRUE_T, widx, fW))
    txmax = jnp.max(jnp.where(ecol > _TRUE_T, widx, -1.0))

    yb[0] = jnp.minimum(jnp.where(first, fH, yb[0]), pymin)
    yb[1] = jnp.maximum(jnp.where(first, -1.0, yb[1]), pymax)
    yb[2] = jnp.minimum(jnp.where(first, fW, yb[2]), pxmin)
    yb[3] = jnp.maximum(jnp.where(first, -1.0, yb[3]), pxmax)
    yb[4] = jnp.minimum(jnp.where(first, fH, yb[4]), tymin)
    yb[5] = jnp.maximum(jnp.where(first, -1.0, yb[5]), tymax)
    yb[6] = jnp.minimum(jnp.where(first, fW, yb[6]), txmin)
    yb[7] = jnp.maximum(jnp.where(first, -1.0, yb[7]), txmax)

    @pl.when(h == nh - 1)
    def _tail():
        def vec(s):
            return jnp.full((1, 128), s, f32)

        py1, py2, px1, px2 = yb[0], yb[1], yb[2], yb[3]
        ty1, ty2, tx1, tx2 = yb[4], yb[5], yb[6], yb[7]
        pa = vec((py2 - py1 + 1.0) * (px2 - px1 + 1.0))
        ta = vec((ty2 - ty1 + 1.0) * (tx2 - tx1 + 1.0))
        area_pen = jnp.maximum(pa - ta, 0.0) / (ta + 1.0)
        cy = vec(py1 + py2) * 0.5 - vec(ty1 + ty2) * 0.5
        cx = vec(px1 + px2) * 0.5 - vec(tx1 + tx2) * 0.5
        center = jnp.sqrt(cy * cy + cx * cx) * (1.0 / 20.0)
        valid = jnp.full((1, 128), (py2 >= 0.0) & (ty2 >= 0.0), jnp.bool_)
        pen = jnp.where(valid, area_pen + center, 1.0)
        prev = jnp.where(b == 0, jnp.zeros_like(pen), psum_ref[...])
        tot = prev + pen
        psum_ref[...] = tot

        @pl.when(b == nb - 1)
        def _():
            out_ref[...] = tot * (_PW / nb)


def kernel(prediction_probs, expected_onehot):
    B, H, W, C = prediction_probs.shape
    pt = prediction_probs.transpose(0, 1, 3, 2)
    et = expected_onehot.transpose(0, 1, 3, 2)
    nh = H // _HB
    out = pl.pallas_call(
        _bbox_body,
        grid=(B, nh),
        in_specs=[
            pl.BlockSpec((1, _HB, C, W), lambda b, h: (b, h, 0, 0)),
            pl.BlockSpec((1, _HB, C, W), lambda b, h: (b, h, 0, 0)),
        ],
        out_specs=pl.BlockSpec((1, 128), lambda b, h: (0, 0)),
        out_shape=jax.ShapeDtypeStruct((1, 128), jnp.float32),
        scratch_shapes=[
            pltpu.SMEM((8,), jnp.float32),
            pltpu.VMEM((1, 128), jnp.float32),
        ],
    )(pt, et)
    return out[0, 0]
